# baseline (speedup 1.0000x reference)
import functools
import numpy as np
import jax
from jax import lax
import jax.numpy as jnp
from jax.experimental import pallas as pl
from jax.experimental.pallas import tpu as pltpu
from jax.experimental.pallas import tpu_sc as plsc

N_NODES = 10000
N_EDGES = 320000
D = 128
DELTA = 2.5
EPS = 1e-5
INV_SQRT_N = 1.0 / np.sqrt(float(N_NODES))

NW = 32
NB = 320
NPAD = NW * NB
ACC_ROWS = 321
BK = 1280
NBLK = N_EDGES // BK
GB = 128
QCAP = 1536
NEG = -3.0e38


def _head_body(x_ref, W_ref, b_ref, Mwa_ref, Mwb_ref, Mb_ref,
               h_ref, a_ref, bmsg_ref):
    h = jnp.dot(x_ref[...], W_ref[...], preferred_element_type=jnp.float32)
    h = h + b_ref[...]
    h_ref[...] = h
    a_ref[...] = jnp.dot(h, Mwa_ref[...], preferred_element_type=jnp.float32)
    bmsg_ref[...] = jnp.dot(h, Mwb_ref[...],
                            preferred_element_type=jnp.float32) + Mb_ref[...]


def _ukern_body(h_ref, S_ref, MX_ref, bmsg_ref, deg_ref,
                Uw_ref, Ub_ref, bng_ref, bnb_ref, y_ref):
    deg = deg_ref[...]
    b = bmsg_ref[...]
    s = S_ref[...] + deg * b
    mean = s / jnp.maximum(deg, 1.0)
    mx = jnp.where(deg > 0, MX_ref[...] + b, 0.0)
    lg = jnp.log(deg + 1.0) * (1.0 / DELTA)
    Uw = Uw_ref[...]
    u = jnp.dot(h_ref[...], Uw[0:D], preferred_element_type=jnp.float32)
    u += jnp.dot(mean, Uw[D:2 * D], preferred_element_type=jnp.float32)
    u += lg * jnp.dot(mean, Uw[4 * D:5 * D],
                      preferred_element_type=jnp.float32)
    u += jnp.dot(mx, Uw[2 * D:3 * D], preferred_element_type=jnp.float32)
    u += lg * jnp.dot(mx, Uw[5 * D:6 * D],
                      preferred_element_type=jnp.float32)
    u += jnp.dot(s, Uw[3 * D:4 * D], preferred_element_type=jnp.float32)
    u += lg * jnp.dot(s, Uw[6 * D:7 * D],
                      preferred_element_type=jnp.float32)
    u = (u + Ub_ref[...]) * INV_SQRT_N
    mu = jnp.mean(u, axis=0, keepdims=True)
    var = jnp.mean((u - mu) * (u - mu), axis=0, keepdims=True)
    y_ref[...] = (u - mu) * jax.lax.rsqrt(var + EPS) * bng_ref[...] \
        + bnb_ref[...]


def _mix_body(y_ref, h_ref, mixw_ref, mixb_ref, out_ref, *maybe_next):
    m = jnp.dot(y_ref[...], mixw_ref[...],
                preferred_element_type=jnp.float32) + mixb_ref[...]
    m = jnp.where(m > 0, m, 0.01 * m)
    hn = m + h_ref[...]
    out_ref[...] = hn
    if maybe_next:
        a_ref, bm_ref, Mwa_ref, Mwb_ref, Mb_ref = maybe_next
        a_ref[...] = jnp.dot(hn, Mwa_ref[...],
                             preferred_element_type=jnp.float32)
        bm_ref[...] = jnp.dot(hn, Mwb_ref[...],
                              preferred_element_type=jnp.float32) + Mb_ref[...]


def _head(x, W_emb, b_emb, Mwa, Mwb, Mb):
    return pl.pallas_call(
        _head_body,
        out_shape=[jax.ShapeDtypeStruct((N_NODES, D), jnp.float32)] * 3,
    )(x, W_emb, b_emb[None, :], Mwa, Mwb, Mb[None, :])


def _tail(h, S, MX, bmsg, deg, Uw, Ub, bng, bnb, mixw, mixb, nxt=None):
    y = pl.pallas_call(
        _ukern_body,
        out_shape=jax.ShapeDtypeStruct((N_NODES, D), jnp.float32),
    )(h, S, MX, bmsg, deg[:, None], Uw, Ub[None, :], bng[None, :],
      bnb[None, :])

    n_out = 1 if nxt is None else 3
    args = [y, h, mixw, mixb[None, :]]
    if nxt is not None:
        Mwa, Mwb, Mb = nxt
        args += [Mwa, Mwb, Mb[None, :]]
    n_in = len(args)

    def body(*refs):
        ins = refs[:n_in]
        outs = refs[n_in:]
        if nxt is None:
            _mix_body(*ins[:4], outs[0])
        else:
            _mix_body(*ins[:4], outs[0], outs[1], outs[2], ins[4], ins[5],
                      ins[6])

    return pl.pallas_call(
        body,
        out_shape=[jax.ShapeDtypeStruct((N_NODES, D), jnp.float32)] * n_out,
    )(*args)


def _sc_body(src_hbm, dst_hbm, a_hbm, S_hbm, MX_hbm, deg_hbm,
             dstbuf0, srcbuf0, dstbuf1, srcbuf1, qpk,
             gbuf0, gbuf1, qs_p0, qs_p1, qd_p0, qd_p1,
             acc_s, acc_m, acc_d,
             sem_s0, sem_s1, sem_g0, sem_g1):
    w = lax.axis_index("s") * 2 + lax.axis_index("c")
    lo = w * NB
    hi = lo + NB
    lane = jnp.arange(16, dtype=jnp.int32)
    zf = jnp.zeros((16,), jnp.float32)
    onesf = jnp.ones((16,), jnp.float32)
    negf = jnp.full((16,), NEG, jnp.float32)

    def zrow(r, carry):
        for sl in range(8):
            acc_s[r, pl.ds(sl * 16, 16)] = zf
            acc_m[r, pl.ds(sl * 16, 16)] = negf
        acc_d[pl.ds(r * 16, 16)] = zf
        return carry

    lax.fori_loop(0, ACC_ROWS, zrow, 0)

    def stage_start(b, dbuf, sbuf, sem):
        pltpu.async_copy(dst_hbm.at[pl.ds(b * BK, BK)], dbuf, sem)
        pltpu.async_copy(src_hbm.at[pl.ds(b * BK, BK)], sbuf, sem)

    def stage_wait(dbuf, sbuf, sem):
        pltpu.make_async_copy(dst_hbm.at[pl.ds(0, BK)], dbuf, sem).wait()
        pltpu.make_async_copy(src_hbm.at[pl.ds(0, BK)], sbuf, sem).wait()

    def fire(off, qs_p, qd_p, gbuf, sem):
        for j in range(GB // 16):
            v = qpk[pl.ds(off + j * 16, 16)]
            qs_p[pl.ds(j * 16, 16)] = v & 16383
            qd_p[pl.ds(j * 16, 16)] = v >> 14
        pltpu.async_copy(a_hbm.at[qs_p], gbuf, sem)

    def drain(qs_p, qd_p, gbuf, sem):
        pltpu.make_async_copy(a_hbm.at[qs_p], gbuf, sem).wait()

        def grp(g16, carry):
            dlv = qd_p[pl.ds(g16 * 16, 16)]
            for e in range(16):
                dl = jnp.sum(jnp.where(lane == e, dlv, 0))
                erow = g16 * 16 + e
                plsc.addupdate(acc_d.at[pl.ds(dl * 16, 16)], onesf)
                for sl in range(8):
                    gv = gbuf[erow, pl.ds(sl * 16, 16)]
                    plsc.addupdate(acc_s.at[dl, pl.ds(sl * 16, 16)], gv)
                    acc_m[dl, pl.ds(sl * 16, 16)] = jnp.maximum(
                        acc_m[dl, pl.ds(sl * 16, 16)], gv)
            return carry

        lax.fori_loop(0, GB // 16, grp, 0)

    def fire_par(fired, off):
        @pl.when(lax.rem(fired, 2) == 0)
        def _():
            fire(off, qs_p0, qd_p0, gbuf0, sem_g0)

        @pl.when(lax.rem(fired, 2) == 1)
        def _():
            fire(off, qs_p1, qd_p1, gbuf1, sem_g1)

    def drain_par(done, cond):
        @pl.when(cond & (lax.rem(done, 2) == 0))
        def _():
            drain(qs_p0, qd_p0, gbuf0, sem_g0)

        @pl.when(cond & (lax.rem(done, 2) == 1))
        def _():
            drain(qs_p1, qd_p1, gbuf1, sem_g1)

        return jnp.where(cond, done + 1, done)

    def scan_chunk(dbuf, sbuf, i, cntv):
        dlv = dbuf[pl.ds(i * 16, 16)]
        srcv = sbuf[pl.ds(i * 16, 16)]
        m = (dlv >= lo) & (dlv < hi)
        mi = m.astype(jnp.int32)
        excl = plsc.cumsum(mi) - mi
        idxv = excl + cntv
        plsc.store_scatter(qpk, [idxv], srcv | ((dlv - lo) << 14), mask=m)
        return cntv + plsc.all_reduce_population_count(m)

    def scan_block(dbuf, sbuf, cntv):
        def two(i, cv):
            cv = scan_chunk(dbuf, sbuf, 2 * i, cv)
            cv = scan_chunk(dbuf, sbuf, 2 * i + 1, cv)
            return cv

        return lax.fori_loop(0, BK // 32, two, cntv)

    def process_block(st):
        cntv, fired, done = st
        cnt = jnp.max(cntv)
        nfull = cnt // GB

        def lp(i, st2):
            fired2, done2 = st2
            done2 = drain_par(done2, fired2 - done2 >= 2)
            fire_par(fired2, i * GB)
            return (fired2 + 1, done2)

        fired, done = lax.fori_loop(0, nfull, lp, (fired, done))

        @pl.when(nfull > 0)
        def _():
            base = nfull * GB
            for j in range(GB // 16):
                qpk[pl.ds(j * 16, 16)] = qpk[pl.ds(base + j * 16, 16)]

        return (cntv - nfull * GB, fired, done)

    def block_pair(p, st):
        b = p * 2
        cntv, fired, done = st
        stage_start(b + 1, dstbuf1, srcbuf1, sem_s1)
        stage_wait(dstbuf0, srcbuf0, sem_s0)
        cntv = scan_block(dstbuf0, srcbuf0, cntv)
        st = process_block((cntv, fired, done))

        @pl.when(b + 2 < NBLK)
        def _():
            stage_start(b + 2, dstbuf0, srcbuf0, sem_s0)

        stage_wait(dstbuf1, srcbuf1, sem_s1)
        cntv, fired, done = st
        cntv = scan_block(dstbuf1, srcbuf1, cntv)
        return process_block((cntv, fired, done))

    stage_start(0, dstbuf0, srcbuf0, sem_s0)
    zi = jnp.zeros((16,), jnp.int32)
    cntv, fired, done = lax.fori_loop(0, NBLK // 2, block_pair,
                                      (zi, jnp.int32(0), jnp.int32(0)))

    cnt = jnp.max(cntv)

    @pl.when(cnt > 0)
    def _():
        for j in range(GB // 16):
            gl = lane + j * 16
            mpad = gl >= cnt
            plsc.store_scatter(qpk, [gl],
                               jnp.full((16,), (ACC_ROWS - 1) << 14,
                                        jnp.int32),
                               mask=mpad)

    done = drain_par(done, (cnt > 0) & (fired - done >= 2))

    @pl.when(cnt > 0)
    def _():
        fire_par(fired, 0)

    fired = jnp.where(cnt > 0, fired + 1, fired)
    done = drain_par(done, done < fired)
    done = drain_par(done, done < fired)

    pltpu.sync_copy(acc_s.at[pl.ds(0, NB)], S_hbm.at[pl.ds(lo, NB)])
    pltpu.sync_copy(acc_m.at[pl.ds(0, NB)], MX_hbm.at[pl.ds(lo, NB)])
    pltpu.sync_copy(acc_d.at[pl.ds(0, NB * 16)],
                    deg_hbm.at[pl.ds(lo * 16, NB * 16)])


_sc_call = functools.partial(
    pl.kernel,
    mesh=plsc.VectorSubcoreMesh(core_axis_name="c", subcore_axis_name="s"),
    compiler_params=pltpu.CompilerParams(needs_layout_passes=False),
    out_type=[
        jax.ShapeDtypeStruct((NPAD, D), jnp.float32),
        jax.ShapeDtypeStruct((NPAD, D), jnp.float32),
        jax.ShapeDtypeStruct((NPAD * 16,), jnp.float32),
    ],
    scratch_types=[
        pltpu.VMEM((BK,), jnp.int32),
        pltpu.VMEM((BK,), jnp.int32),
        pltpu.VMEM((BK,), jnp.int32),
        pltpu.VMEM((BK,), jnp.int32),
        pltpu.VMEM((QCAP,), jnp.int32),
        pltpu.VMEM((GB, D), jnp.float32),
        pltpu.VMEM((GB, D), jnp.float32),
        pltpu.VMEM((GB,), jnp.int32),
        pltpu.VMEM((GB,), jnp.int32),
        pltpu.VMEM((GB,), jnp.int32),
        pltpu.VMEM((GB,), jnp.int32),
        pltpu.VMEM((ACC_ROWS, D), jnp.float32),
        pltpu.VMEM((ACC_ROWS, D), jnp.float32),
        pltpu.VMEM((ACC_ROWS * 16,), jnp.float32),
        pltpu.SemaphoreType.DMA,
        pltpu.SemaphoreType.DMA,
        pltpu.SemaphoreType.DMA,
        pltpu.SemaphoreType.DMA,
    ],
)


def _segments(a, src, dst):
    S, MX, deg = _sc_call(_sc_body)(src, dst, a)
    return S[:N_NODES], MX[:N_NODES], deg.reshape(NPAD, 16)[:N_NODES, 0]


def kernel(x, edge_index, W_emb, b_emb, M_w1, M_b1, U_w1, U_b1, bn_g1, bn_b1,
           mix_w1, mix_b1, M_w2, M_b2, U_w2, U_b2, bn_g2, bn_b2, mix_w2,
           mix_b2):
    src = edge_index[0]
    dst = edge_index[1]

    h, a1, b1 = _head(x, W_emb, b_emb, M_w1[:D], M_w1[D:], M_b1)
    S1, MX1, deg = _segments(a1, src, dst)
    h2, a2, b2 = _tail(h, S1, MX1, b1, deg, U_w1, U_b1, bn_g1, bn_b1,
                       mix_w1, mix_b1, nxt=(M_w2[:D], M_w2[D:], M_b2))
    S2, MX2, _ = _segments(a2, src, dst)
    h3 = _tail(h2, S2, MX2, b2, deg, U_w2, U_b2, bn_g2, bn_b2,
               mix_w2, mix_b2)[0]
    return h3


# speedup vs baseline: 1.0256x; 1.0256x over previous
import functools
import numpy as np
import jax
from jax import lax
import jax.numpy as jnp
from jax.experimental import pallas as pl
from jax.experimental.pallas import tpu as pltpu
from jax.experimental.pallas import tpu_sc as plsc

N_NODES = 10000
N_EDGES = 320000
D = 128
DELTA = 2.5
EPS = 1e-5
INV_SQRT_N = 1.0 / np.sqrt(float(N_NODES))

NW = 32
NB = 320
NPAD = NW * NB
ACC_ROWS = 321
BK = 1280
NBLK = N_EDGES // BK
GB = 64
QCAP = 1536
NEG = -3.0e38


def _head_body(x_ref, W_ref, b_ref, Mwa_ref, Mwb_ref, Mb_ref,
               h_ref, a_ref, bmsg_ref):
    h = jnp.dot(x_ref[...], W_ref[...], preferred_element_type=jnp.float32)
    h = h + b_ref[...]
    h_ref[...] = h
    a_ref[...] = jnp.dot(h, Mwa_ref[...], preferred_element_type=jnp.float32)
    bmsg_ref[...] = jnp.dot(h, Mwb_ref[...],
                            preferred_element_type=jnp.float32) + Mb_ref[...]


def _ukern_body(h_ref, S_ref, MX_ref, bmsg_ref, deg_ref,
                Uw_ref, Ub_ref, bng_ref, bnb_ref, y_ref):
    deg = deg_ref[...]
    b = bmsg_ref[...]
    s = S_ref[...] + deg * b
    mean = s / jnp.maximum(deg, 1.0)
    mx = jnp.where(deg > 0, MX_ref[...] + b, 0.0)
    lg = jnp.log(deg + 1.0) * (1.0 / DELTA)
    Uw = Uw_ref[...]
    u = jnp.dot(h_ref[...], Uw[0:D], preferred_element_type=jnp.float32)
    u += jnp.dot(mean, Uw[D:2 * D], preferred_element_type=jnp.float32)
    u += lg * jnp.dot(mean, Uw[4 * D:5 * D],
                      preferred_element_type=jnp.float32)
    u += jnp.dot(mx, Uw[2 * D:3 * D], preferred_element_type=jnp.float32)
    u += lg * jnp.dot(mx, Uw[5 * D:6 * D],
                      preferred_element_type=jnp.float32)
    u += jnp.dot(s, Uw[3 * D:4 * D], preferred_element_type=jnp.float32)
    u += lg * jnp.dot(s, Uw[6 * D:7 * D],
                      preferred_element_type=jnp.float32)
    u = (u + Ub_ref[...]) * INV_SQRT_N
    mu = jnp.mean(u, axis=0, keepdims=True)
    var = jnp.mean((u - mu) * (u - mu), axis=0, keepdims=True)
    y_ref[...] = (u - mu) * jax.lax.rsqrt(var + EPS) * bng_ref[...] \
        + bnb_ref[...]


def _mix_body(y_ref, h_ref, mixw_ref, mixb_ref, out_ref, *maybe_next):
    m = jnp.dot(y_ref[...], mixw_ref[...],
                preferred_element_type=jnp.float32) + mixb_ref[...]
    m = jnp.where(m > 0, m, 0.01 * m)
    hn = m + h_ref[...]
    out_ref[...] = hn
    if maybe_next:
        a_ref, bm_ref, Mwa_ref, Mwb_ref, Mb_ref = maybe_next
        a_ref[...] = jnp.dot(hn, Mwa_ref[...],
                             preferred_element_type=jnp.float32)
        bm_ref[...] = jnp.dot(hn, Mwb_ref[...],
                              preferred_element_type=jnp.float32) + Mb_ref[...]


def _head(x, W_emb, b_emb, Mwa, Mwb, Mb):
    return pl.pallas_call(
        _head_body,
        out_shape=[jax.ShapeDtypeStruct((N_NODES, D), jnp.float32)] * 3,
    )(x, W_emb, b_emb[None, :], Mwa, Mwb, Mb[None, :])


def _tail(h, S, MX, bmsg, deg, Uw, Ub, bng, bnb, mixw, mixb, nxt=None):
    y = pl.pallas_call(
        _ukern_body,
        out_shape=jax.ShapeDtypeStruct((N_NODES, D), jnp.float32),
    )(h, S, MX, bmsg, deg[:, None], Uw, Ub[None, :], bng[None, :],
      bnb[None, :])

    n_out = 1 if nxt is None else 3
    args = [y, h, mixw, mixb[None, :]]
    if nxt is not None:
        Mwa, Mwb, Mb = nxt
        args += [Mwa, Mwb, Mb[None, :]]
    n_in = len(args)

    def body(*refs):
        ins = refs[:n_in]
        outs = refs[n_in:]
        if nxt is None:
            _mix_body(*ins[:4], outs[0])
        else:
            _mix_body(*ins[:4], outs[0], outs[1], outs[2], ins[4], ins[5],
                      ins[6])

    return pl.pallas_call(
        body,
        out_shape=[jax.ShapeDtypeStruct((N_NODES, D), jnp.float32)] * n_out,
    )(*args)


def _sc_body(src_hbm, dst_hbm, a_hbm, S_hbm, MX_hbm, deg_hbm,
             dstbuf0, srcbuf0, dstbuf1, srcbuf1, qpk,
             gbuf0, gbuf1, qs_p0, qs_p1, qd_p0, qd_p1,
             acc_s, acc_m, acc_d,
             sem_s0, sem_s1, sem_g0, sem_g1):
    w = lax.axis_index("s") * 2 + lax.axis_index("c")
    lo = w * NB
    hi = lo + NB
    lane = jnp.arange(16, dtype=jnp.int32)
    zf = jnp.zeros((16,), jnp.float32)
    onesf = jnp.ones((16,), jnp.float32)
    negf = jnp.full((16,), NEG, jnp.float32)

    def zrow(r, carry):
        for sl in range(8):
            acc_s[r, pl.ds(sl * 16, 16)] = zf
            acc_m[r, pl.ds(sl * 16, 16)] = negf
        acc_d[pl.ds(r * 16, 16)] = zf
        return carry

    lax.fori_loop(0, ACC_ROWS, zrow, 0)

    def stage_start(b, dbuf, sbuf, sem):
        pltpu.async_copy(dst_hbm.at[pl.ds(b * BK, BK)], dbuf, sem)
        pltpu.async_copy(src_hbm.at[pl.ds(b * BK, BK)], sbuf, sem)

    def stage_wait(dbuf, sbuf, sem):
        pltpu.make_async_copy(dst_hbm.at[pl.ds(0, BK)], dbuf, sem).wait()
        pltpu.make_async_copy(src_hbm.at[pl.ds(0, BK)], sbuf, sem).wait()

    def fire(off, qs_p, qd_p, gbuf, sem):
        for j in range(GB // 16):
            v = qpk[pl.ds(off + j * 16, 16)]
            qs_p[pl.ds(j * 16, 16)] = v & 16383
            qd_p[pl.ds(j * 16, 16)] = v >> 14
        pltpu.async_copy(a_hbm.at[qs_p], gbuf, sem)

    def drain(qs_p, qd_p, gbuf, sem):
        pltpu.make_async_copy(a_hbm.at[qs_p], gbuf, sem).wait()

        def grp(g16, carry):
            dlv = qd_p[pl.ds(g16 * 16, 16)]
            for e in range(16):
                dl = jnp.sum(jnp.where(lane == e, dlv, 0))
                erow = g16 * 16 + e
                plsc.addupdate(acc_d.at[pl.ds(dl * 16, 16)], onesf)
                for sl in range(8):
                    gv = gbuf[erow, pl.ds(sl * 16, 16)]
                    plsc.addupdate(acc_s.at[dl, pl.ds(sl * 16, 16)], gv)
                    acc_m[dl, pl.ds(sl * 16, 16)] = jnp.maximum(
                        acc_m[dl, pl.ds(sl * 16, 16)], gv)
            return carry

        lax.fori_loop(0, GB // 16, grp, 0)

    def fire_par(fired, off):
        @pl.when(lax.rem(fired, 2) == 0)
        def _():
            fire(off, qs_p0, qd_p0, gbuf0, sem_g0)

        @pl.when(lax.rem(fired, 2) == 1)
        def _():
            fire(off, qs_p1, qd_p1, gbuf1, sem_g1)

    def drain_par(done, cond):
        @pl.when(cond & (lax.rem(done, 2) == 0))
        def _():
            drain(qs_p0, qd_p0, gbuf0, sem_g0)

        @pl.when(cond & (lax.rem(done, 2) == 1))
        def _():
            drain(qs_p1, qd_p1, gbuf1, sem_g1)

        return jnp.where(cond, done + 1, done)

    def scan_chunk(dbuf, sbuf, i, cntv):
        dlv = dbuf[pl.ds(i * 16, 16)]
        srcv = sbuf[pl.ds(i * 16, 16)]
        m = (dlv >= lo) & (dlv < hi)
        mi = m.astype(jnp.int32)
        excl = plsc.cumsum(mi) - mi
        idxv = excl + cntv
        plsc.store_scatter(qpk, [idxv], srcv | ((dlv - lo) << 14), mask=m)
        return cntv + plsc.all_reduce_population_count(m)

    def scan_block(dbuf, sbuf, cntv):
        def two(i, cv):
            cv = scan_chunk(dbuf, sbuf, 2 * i, cv)
            cv = scan_chunk(dbuf, sbuf, 2 * i + 1, cv)
            return cv

        return lax.fori_loop(0, BK // 32, two, cntv)

    def process_block(st):
        cntv, fired, done = st
        cnt = jnp.max(cntv)
        nfull = cnt // GB

        def lp(i, st2):
            fired2, done2 = st2
            done2 = drain_par(done2, fired2 - done2 >= 2)
            fire_par(fired2, i * GB)
            return (fired2 + 1, done2)

        fired, done = lax.fori_loop(0, nfull, lp, (fired, done))

        @pl.when(nfull > 0)
        def _():
            base = nfull * GB
            for j in range(GB // 16):
                qpk[pl.ds(j * 16, 16)] = qpk[pl.ds(base + j * 16, 16)]

        return (cntv - nfull * GB, fired, done)

    def block_pair(p, st):
        b = p * 2
        cntv, fired, done = st
        stage_start(b + 1, dstbuf1, srcbuf1, sem_s1)
        stage_wait(dstbuf0, srcbuf0, sem_s0)
        cntv = scan_block(dstbuf0, srcbuf0, cntv)
        st = process_block((cntv, fired, done))

        @pl.when(b + 2 < NBLK)
        def _():
            stage_start(b + 2, dstbuf0, srcbuf0, sem_s0)

        stage_wait(dstbuf1, srcbuf1, sem_s1)
        cntv, fired, done = st
        cntv = scan_block(dstbuf1, srcbuf1, cntv)
        return process_block((cntv, fired, done))

    stage_start(0, dstbuf0, srcbuf0, sem_s0)
    zi = jnp.zeros((16,), jnp.int32)
    cntv, fired, done = lax.fori_loop(0, NBLK // 2, block_pair,
                                      (zi, jnp.int32(0), jnp.int32(0)))

    cnt = jnp.max(cntv)

    @pl.when(cnt > 0)
    def _():
        for j in range(GB // 16):
            gl = lane + j * 16
            mpad = gl >= cnt
            plsc.store_scatter(qpk, [gl],
                               jnp.full((16,), (ACC_ROWS - 1) << 14,
                                        jnp.int32),
                               mask=mpad)

    done = drain_par(done, (cnt > 0) & (fired - done >= 2))

    @pl.when(cnt > 0)
    def _():
        fire_par(fired, 0)

    fired = jnp.where(cnt > 0, fired + 1, fired)
    done = drain_par(done, done < fired)
    done = drain_par(done, done < fired)

    pltpu.sync_copy(acc_s.at[pl.ds(0, NB)], S_hbm.at[pl.ds(lo, NB)])
    pltpu.sync_copy(acc_m.at[pl.ds(0, NB)], MX_hbm.at[pl.ds(lo, NB)])
    pltpu.sync_copy(acc_d.at[pl.ds(0, NB * 16)],
                    deg_hbm.at[pl.ds(lo * 16, NB * 16)])


_sc_call = functools.partial(
    pl.kernel,
    mesh=plsc.VectorSubcoreMesh(core_axis_name="c", subcore_axis_name="s"),
    compiler_params=pltpu.CompilerParams(needs_layout_passes=False),
    out_type=[
        jax.ShapeDtypeStruct((NPAD, D), jnp.float32),
        jax.ShapeDtypeStruct((NPAD, D), jnp.float32),
        jax.ShapeDtypeStruct((NPAD * 16,), jnp.float32),
    ],
    scratch_types=[
        pltpu.VMEM((BK,), jnp.int32),
        pltpu.VMEM((BK,), jnp.int32),
        pltpu.VMEM((BK,), jnp.int32),
        pltpu.VMEM((BK,), jnp.int32),
        pltpu.VMEM((QCAP,), jnp.int32),
        pltpu.VMEM((GB, D), jnp.float32),
        pltpu.VMEM((GB, D), jnp.float32),
        pltpu.VMEM((GB,), jnp.int32),
        pltpu.VMEM((GB,), jnp.int32),
        pltpu.VMEM((GB,), jnp.int32),
        pltpu.VMEM((GB,), jnp.int32),
        pltpu.VMEM((ACC_ROWS, D), jnp.float32),
        pltpu.VMEM((ACC_ROWS, D), jnp.float32),
        pltpu.VMEM((ACC_ROWS * 16,), jnp.float32),
        pltpu.SemaphoreType.DMA,
        pltpu.SemaphoreType.DMA,
        pltpu.SemaphoreType.DMA,
        pltpu.SemaphoreType.DMA,
    ],
)


def _segments(a, src, dst):
    S, MX, deg = _sc_call(_sc_body)(src, dst, a)
    return S[:N_NODES], MX[:N_NODES], deg.reshape(NPAD, 16)[:N_NODES, 0]


def kernel(x, edge_index, W_emb, b_emb, M_w1, M_b1, U_w1, U_b1, bn_g1, bn_b1,
           mix_w1, mix_b1, M_w2, M_b2, U_w2, U_b2, bn_g2, bn_b2, mix_w2,
           mix_b2):
    src = edge_index[0]
    dst = edge_index[1]

    h, a1, b1 = _head(x, W_emb, b_emb, M_w1[:D], M_w1[D:], M_b1)
    S1, MX1, deg = _segments(a1, src, dst)
    h2, a2, b2 = _tail(h, S1, MX1, b1, deg, U_w1, U_b1, bn_g1, bn_b1,
                       mix_w1, mix_b1, nxt=(M_w2[:D], M_w2[D:], M_b2))
    S2, MX2, _ = _segments(a2, src, dst)
    h3 = _tail(h2, S2, MX2, b2, deg, U_w2, U_b2, bn_g2, bn_b2,
               mix_w2, mix_b2)[0]
    return h3
